# Initial kernel scaffold; baseline (speedup 1.0000x reference)
import jax
import jax.numpy as jnp
from jax.experimental import pallas as pl


def kernel(rfs, ids, samples_idx):
    raise NotImplementedError("write your pallas kernel here")


# speedup vs baseline: 921.2444x; 921.2444x over previous
import functools
import jax
import jax.numpy as jnp
from jax import lax
from jax.experimental import pallas as pl
from jax.experimental.pallas import tpu as pltpu
from jax.experimental.pallas import tpu_sc as plsc

_B, _K, _NC, _NS = 8, 2, 16, 2048
_NP = 256 * 128
_NW = 32
_TPB = _NW // _B
_PPT = _NP // _TPB
_CP = 2048
_NJ = _PPT // _CP
_NJT = _NP // _CP
_KOFF = _NC * _NS


def _sc_das(rfs_flat, samples_flat, rowidx):
    mesh = plsc.VectorSubcoreMesh(core_axis_name="core", subcore_axis_name="subcore")

    @functools.partial(
        pl.kernel,
        mesh=mesh,
        out_type=jax.ShapeDtypeStruct((_B, _NP * _K), jnp.float32),
        compiler_params=pltpu.CompilerParams(needs_layout_passes=False),
        scratch_types=[
            pltpu.VMEM((_K * _NC * _NS,), jnp.float32),
            pltpu.VMEM((_NC, _CP), jnp.float32),
            pltpu.VMEM((2 * _CP,), jnp.float32),
            pltpu.VMEM((_NJ, _NC), jnp.int32),
            pltpu.SemaphoreType.DMA,
        ],
    )
    def k(rfs_hbm, samples_hbm, rowidx_hbm, out_hbm, rfs_v, idx_v, acc_v, row_v, sem):
        cid = lax.axis_index("core")
        sid = lax.axis_index("subcore")
        wid = sid * 2 + cid
        b = wid // _TPB
        q = wid % _TPB
        pltpu.sync_copy(rowidx_hbm.at[wid], row_v)
        pltpu.sync_copy(rfs_hbm.at[b], rfs_v)
        iota2 = lax.iota(jnp.int32, 16) * 2

        def jbody(j, carry):
            pltpu.async_copy(samples_hbm.at[row_v.at[j]], idx_v, sem).wait()

            def gbody(g, carry2):
                pbase = g * 16
                acc0 = jnp.zeros((16,), jnp.float32)
                acc1 = jnp.zeros((16,), jnp.float32)
                for c in range(_NC):
                    fi = idx_v[c, pl.ds(pbase, 16)]
                    i0 = fi.astype(jnp.int32)
                    w = fi - i0.astype(jnp.float32)
                    i1 = jnp.minimum(i0 + 1, _NS - 1)
                    a0 = i0 + c * _NS
                    a1 = i1 + c * _NS
                    v00 = plsc.load_gather(rfs_v, [a0])
                    v01 = plsc.load_gather(rfs_v, [a1])
                    v10 = plsc.load_gather(rfs_v, [a0 + _KOFF])
                    v11 = plsc.load_gather(rfs_v, [a1 + _KOFF])
                    acc0 = acc0 + v00 + w * (v01 - v00)
                    acc1 = acc1 + v10 + w * (v11 - v10)
                pos = pbase * 2 + iota2
                plsc.store_scatter(acc_v, [pos], acc0)
                plsc.store_scatter(acc_v, [pos + 1], acc1)
                return carry2

            lax.fori_loop(0, _CP // 16, gbody, 0)
            ostart = (q * _NJ + j) * (2 * _CP)
            pltpu.sync_copy(acc_v, out_hbm.at[b, pl.ds(ostart, 2 * _CP)])
            return carry

        lax.fori_loop(0, _NJ, jbody, 0)

    return k(rfs_flat, samples_flat, rowidx)


def kernel(rfs, ids, samples_idx):
    b, kk, nc, ns = rfs.shape
    s, _, nz, nx = samples_idx.shape
    rfs_flat = rfs.reshape(b, kk * nc * ns)
    samples_flat = samples_idx.reshape(s * nc * _NJT, _CP)
    w = jnp.arange(_NW, dtype=jnp.int32)
    jj = jnp.arange(_NJ, dtype=jnp.int32)
    cc = jnp.arange(nc, dtype=jnp.int32)
    bw = w // _TPB
    qw = w % _TPB
    rows = (ids[bw][:, None, None] * nc + cc[None, None, :]) * _NJT \
        + (qw[:, None] * _NJ + jj[None, :])[:, :, None]
    out = _sc_das(rfs_flat, samples_flat, rows.astype(jnp.int32))
    return out.reshape(b, nz, nx, kk)


# speedup vs baseline: 1016.5711x; 1.1035x over previous
import functools
import jax
import jax.numpy as jnp
from jax import lax
from jax.experimental import pallas as pl
from jax.experimental.pallas import tpu as pltpu
from jax.experimental.pallas import tpu_sc as plsc

_B, _K, _NC, _NS = 8, 2, 16, 2048
_NP = 256 * 128
_NW = 32
_TPB = _NW // _B
_PPT = _NP // _TPB
_CP = 2048
_NJ = _PPT // _CP
_NJT = _NP // _CP
_KOFF = _NC * _NS


def _sc_das(rfs_flat, samples_flat, rowidx):
    mesh = plsc.VectorSubcoreMesh(core_axis_name="core", subcore_axis_name="subcore")

    @functools.partial(
        pl.kernel,
        mesh=mesh,
        out_type=jax.ShapeDtypeStruct((_B, _NP * _K), jnp.float32),
        compiler_params=pltpu.CompilerParams(needs_layout_passes=False),
        scratch_types=[
            pltpu.VMEM((_NC * _NS + 16,), jnp.float32),
            pltpu.VMEM((_NC * _NS + 16,), jnp.float32),
            pltpu.VMEM((_NC, _CP), jnp.float32),
            pltpu.VMEM((2 * _CP,), jnp.float32),
            pltpu.VMEM((_NJ, _NC), jnp.int32),
            pltpu.SemaphoreType.DMA,
        ],
    )
    def k(rfs_hbm, samples_hbm, rowidx_hbm, out_hbm, rfs0_v, rfs1_v, idx_v, acc_v,
          row_v, sem):
        cid = lax.axis_index("core")
        sid = lax.axis_index("subcore")
        wid = sid * 2 + cid
        b = wid // _TPB
        q = wid % _TPB
        pltpu.sync_copy(rowidx_hbm.at[wid], row_v)
        pltpu.sync_copy(rfs_hbm.at[b, pl.ds(0, _KOFF)], rfs0_v.at[pl.ds(0, _KOFF)])
        pltpu.sync_copy(rfs_hbm.at[b, pl.ds(_KOFF, _KOFF)], rfs1_v.at[pl.ds(0, _KOFF)])
        zero16 = jnp.zeros((16,), jnp.float32)
        rfs0_v[pl.ds(_KOFF, 16)] = zero16
        rfs1_v[pl.ds(_KOFF, 16)] = zero16
        iota2 = lax.iota(jnp.int32, 16) * 2

        def jbody(j, carry):
            pltpu.async_copy(samples_hbm.at[row_v.at[j]], idx_v, sem).wait()

            @plsc.parallel_loop(0, _CP // 16)
            def gbody(g):
                pbase = g * 16
                acc0 = jnp.zeros((16,), jnp.float32)
                acc1 = jnp.zeros((16,), jnp.float32)
                for c in range(_NC):
                    fi = idx_v[c, pl.ds(pbase, 16)]
                    i0 = fi.astype(jnp.int32)
                    w = fi - i0.astype(jnp.float32)
                    a0 = i0 + c * _NS
                    a1 = a0 + 1
                    v00 = plsc.load_gather(rfs0_v, [a0])
                    v01 = plsc.load_gather(rfs0_v, [a1])
                    v10 = plsc.load_gather(rfs1_v, [a0])
                    v11 = plsc.load_gather(rfs1_v, [a1])
                    acc0 = acc0 + v00 + w * (v01 - v00)
                    acc1 = acc1 + v10 + w * (v11 - v10)
                pos = pbase * 2 + iota2
                plsc.store_scatter(acc_v, [pos], acc0)
                plsc.store_scatter(acc_v, [pos + 1], acc1)

            ostart = (q * _NJ + j) * (2 * _CP)
            pltpu.sync_copy(acc_v, out_hbm.at[b, pl.ds(ostart, 2 * _CP)])
            return carry

        lax.fori_loop(0, _NJ, jbody, 0)

    return k(rfs_flat, samples_flat, rowidx)


def kernel(rfs, ids, samples_idx):
    b, kk, nc, ns = rfs.shape
    s, _, nz, nx = samples_idx.shape
    rfs_flat = rfs.reshape(b, kk * nc * ns)
    samples_flat = samples_idx.reshape(s * nc * _NJT, _CP)
    w = jnp.arange(_NW, dtype=jnp.int32)
    jj = jnp.arange(_NJ, dtype=jnp.int32)
    cc = jnp.arange(nc, dtype=jnp.int32)
    bw = w // _TPB
    qw = w % _TPB
    rows = (ids[bw][:, None, None] * nc + cc[None, None, :]) * _NJT \
        + (qw[:, None] * _NJ + jj[None, :])[:, :, None]
    out = _sc_das(rfs_flat, samples_flat, rows.astype(jnp.int32))
    return out.reshape(b, nz, nx, kk)


# speedup vs baseline: 1586.8142x; 1.5609x over previous
import functools
import jax
import jax.numpy as jnp
from jax import lax
from jax.experimental import pallas as pl
from jax.experimental.pallas import tpu as pltpu
from jax.experimental.pallas import tpu_sc as plsc

_B, _K, _NC, _NS = 8, 2, 16, 2048
_NZ, _NX = 256, 128
_NP = _NZ * _NX
_NW = 32
_TPB = _NW // _B
_PPT = _NP // _TPB
_CP = 2048
_ZC = _CP // _NX
_NJ = _PPT // _CP
_PLANE = _NC * _NS


def _sc_das(rfs, samples_flat, rowidx):
    mesh = plsc.VectorSubcoreMesh(core_axis_name="core", subcore_axis_name="subcore")

    @functools.partial(
        pl.kernel,
        mesh=mesh,
        out_type=jax.ShapeDtypeStruct((_B, _NZ, _K, _NX), jnp.float32),
        compiler_params=pltpu.CompilerParams(needs_layout_passes=False),
        scratch_types=[
            pltpu.VMEM((_PLANE + 16,), jnp.float32),
            pltpu.VMEM((_PLANE + 16,), jnp.float32),
            pltpu.VMEM((_NC * _ZC, _NX), jnp.float32),
            pltpu.VMEM((_ZC, _K, _NX), jnp.float32),
            pltpu.VMEM((_NJ, 2, 128), jnp.int32),
            pltpu.SemaphoreType.DMA,
        ],
    )
    def k(rfs_hbm, samples_hbm, rowidx_hbm, out_hbm, rfs0_v, rfs1_v, idx_v, acc_v,
          row_v, sem):
        cid = lax.axis_index("core")
        sid = lax.axis_index("subcore")
        wid = sid * 2 + cid
        b = wid // _TPB
        q = wid % _TPB
        pltpu.sync_copy(rowidx_hbm.at[wid], row_v)
        copies = []
        for kk, dst in ((0, rfs0_v), (1, rfs1_v)):
            for c in range(_NC):
                copies.append(pltpu.async_copy(
                    rfs_hbm.at[b, kk, c], dst.at[pl.ds(c * _NS, _NS)], sem))
        for cp in copies:
            cp.wait()
        zero16 = jnp.zeros((16,), jnp.float32)
        rfs0_v[pl.ds(_PLANE, 16)] = zero16
        rfs1_v[pl.ds(_PLANE, 16)] = zero16

        def jbody(j, carry):
            c1 = pltpu.async_copy(
                samples_hbm.at[row_v.at[j, 0]], idx_v.at[pl.ds(0, 128)], sem)
            c2 = pltpu.async_copy(
                samples_hbm.at[row_v.at[j, 1]], idx_v.at[pl.ds(128, 128)], sem)
            c1.wait()
            c2.wait()

            @plsc.parallel_loop(0, _CP // 16)
            def gbody(g):
                zloc = g // 8
                xbase = (g % 8) * 16
                acc0 = jnp.zeros((16,), jnp.float32)
                acc1 = jnp.zeros((16,), jnp.float32)
                for c in range(_NC):
                    fi = idx_v[c * _ZC + zloc, pl.ds(xbase, 16)]
                    i0 = fi.astype(jnp.int32)
                    w = fi - i0.astype(jnp.float32)
                    a0 = i0 + c * _NS
                    a1 = a0 + 1
                    v00 = plsc.load_gather(rfs0_v, [a0])
                    v01 = plsc.load_gather(rfs0_v, [a1])
                    v10 = plsc.load_gather(rfs1_v, [a0])
                    v11 = plsc.load_gather(rfs1_v, [a1])
                    acc0 = acc0 + v00 + w * (v01 - v00)
                    acc1 = acc1 + v10 + w * (v11 - v10)
                acc_v[zloc, 0, pl.ds(xbase, 16)] = acc0
                acc_v[zloc, 1, pl.ds(xbase, 16)] = acc1

            z0 = q * (_NJ * _ZC) + j * _ZC
            pltpu.sync_copy(acc_v, out_hbm.at[b, pl.ds(z0, _ZC)])
            return carry

        lax.fori_loop(0, _NJ, jbody, 0)

    return k(rfs, samples_flat, rowidx)


def kernel(rfs, ids, samples_idx):
    b, kk, nc, ns = rfs.shape
    s, _, nz, nx = samples_idx.shape
    samples_flat = samples_idx.reshape(s * nc * nz, nx)
    w = jnp.arange(_NW, dtype=jnp.int32)
    jj = jnp.arange(_NJ, dtype=jnp.int32)
    hh = jnp.arange(2, dtype=jnp.int32)
    dd = jnp.arange(128, dtype=jnp.int32)
    cch = hh[:, None] * 8 + dd[None, :] // _ZC
    zz = dd % _ZC
    bw = w // _TPB
    qw = w % _TPB
    z0 = qw[:, None] * (_NJ * _ZC) + jj[None, :] * _ZC
    rows = ((ids[bw][:, None, None, None] * nc + cch[None, None]) * nz
            + z0[:, :, None, None] + zz[None, None, None, :])
    out = _sc_das(rfs, samples_flat, rows.astype(jnp.int32))
    return out.transpose(0, 1, 3, 2)
